# baseline (speedup 1.0000x reference)
import functools

import jax
import jax.numpy as jnp
from jax import lax
from jax.experimental import pallas as pl
from jax.experimental.pallas import tpu as pltpu, tpu_sc as plsc

N = 10000
E = 320000
D = 128
NC = 2
NS = 16
NW = NC * NS
E_PER_TILE = E // NW
KB = 128
NCH = 80
PAD = NCH * KB - E_PER_TILE
NACC = N + 8
NPS = 624
NREM = N - NS * NPS
ZREM = NACC - NS * NPS

_HIGH = jax.lax.Precision.HIGHEST


def _zero_acc(s, zrow_hbm, acc):
    pltpu.sync_copy(zrow_hbm, acc.at[pl.ds(s * NPS, NPS)])

    @pl.when(s == NS - 1)
    def _zero_rem():
        pltpu.sync_copy(zrow_hbm.at[pl.ds(0, ZREM)],
                        acc.at[pl.ds(NS * NPS, ZREM)])


def _readout(c, s, acc, s_out):
    pltpu.sync_copy(acc.at[pl.ds(s * NPS, NPS)],
                    s_out.at[c, pl.ds(s * NPS, NPS)])

    @pl.when(s == NS - 1)
    def _out_rem():
        pltpu.sync_copy(acc.at[pl.ds(N - NREM, NREM)],
                        s_out.at[c, pl.ds(N - NREM, NREM)])


IDXB = NCH // 2


def _agg_body(y_hbm, src_hbm, dst_hbm, zrow_hbm, s_out,
              src_v, dst_v, rows0, acc):
    c = lax.axis_index("c")
    s = lax.axis_index("s")
    wid = c * NS + s

    _zero_acc(s, zrow_hbm, acc)
    plsc.subcore_barrier()

    for p in range(NCH // IDXB):
        pltpu.sync_copy(src_hbm.at[wid, pl.ds(p * IDXB, IDXB)], src_v)
        pltpu.sync_copy(dst_hbm.at[wid, pl.ds(p * IDXB, IDXB)], dst_v)

        def step(j, t):
            pltpu.sync_copy(y_hbm.at[src_v.at[j]], rows0)
            pltpu.sync_copy(rows0, acc.at[dst_v.at[j]], add=True)
            return t

        lax.fori_loop(0, IDXB, step, 0)

    plsc.subcore_barrier()
    _readout(c, s, acc, s_out)


def _deg_body(ones_hbm, dst_hbm, zrow_hbm, deg_out,
              dst_v, rows0, acc):
    c = lax.axis_index("c")
    s = lax.axis_index("s")
    wid = c * NS + s

    pltpu.sync_copy(ones_hbm, rows0)
    _zero_acc(s, zrow_hbm, acc)
    plsc.subcore_barrier()

    for p in range(NCH // IDXB):
        pltpu.sync_copy(dst_hbm.at[wid, pl.ds(p * IDXB, IDXB)], dst_v)

        def step(j, t):
            pltpu.sync_copy(rows0, acc.at[dst_v.at[j]], add=True)
            return t

        lax.fori_loop(0, IDXB, step, 0)

    plsc.subcore_barrier()
    _readout(c, s, acc, deg_out)


@functools.cache
def _sc_kernels():
    mesh = plsc.VectorSubcoreMesh(
        core_axis_name="c", subcore_axis_name="s",
        num_cores=NC, num_subcores=NS)
    agg = pl.kernel(
        _agg_body,
        out_type=jax.ShapeDtypeStruct((NC, N, D), jnp.float32),
        mesh=mesh,
        scratch_types=[
            pltpu.VMEM((IDXB, KB), jnp.int32),
            pltpu.VMEM((IDXB, KB), jnp.int32),
            pltpu.VMEM((KB, D), jnp.float32),
            pltpu.VMEM_SHARED((NACC, D), jnp.float32),
        ],
    )
    deg = pl.kernel(
        _deg_body,
        out_type=jax.ShapeDtypeStruct((NC, N, D), jnp.float32),
        mesh=mesh,
        scratch_types=[
            pltpu.VMEM((IDXB, KB), jnp.int32),
            pltpu.VMEM((KB, D), jnp.float32),
            pltpu.VMEM_SHARED((NACC, D), jnp.float32),
        ],
    )
    return agg, deg


R = 2000
GRID = N // R


def _dual_mm_body(x_ref, w1_ref, w2_ref, y1_ref, y2_ref):
    xb = x_ref[...]
    y1_ref[...] = jnp.dot(xb, w1_ref[...], precision=_HIGH,
                          preferred_element_type=jnp.float32)
    y2_ref[...] = jnp.dot(xb, w2_ref[...], precision=_HIGH,
                          preferred_element_type=jnp.float32)


def _inv_deg(dg_ref):
    deg = dg_ref[0] + dg_ref[1]
    return 1.0 / jnp.maximum(deg[:, 0:1], 1.0)


def _combine_ba_body(relu_g, has_res, s_ref, dg_ref, h_ref, wr_ref, b_ref,
                     wln_ref, out_ref, y_ref):
    h = h_ref[...]
    g = jnp.maximum(h, 0.0) if relu_g else h
    out = ((s_ref[0] + s_ref[1]) * _inv_deg(dg_ref)
           + jnp.dot(g, wr_ref[...], precision=_HIGH,
                     preferred_element_type=jnp.float32)
           + b_ref[...])
    if has_res:
        out = out + h
    out_ref[...] = out
    y_ref[...] = jnp.dot(jnp.maximum(out, 0.0), wln_ref[...], precision=_HIGH,
                         preferred_element_type=jnp.float32)


def _combine_last_body(s_ref, dg_ref, h_ref, wr_ref, b_ref, out_ref):
    h = h_ref[...]
    out_ref[...] = ((s_ref[0] + s_ref[1]) * _inv_deg(dg_ref)
                    + jnp.dot(jnp.maximum(h, 0.0), wr_ref[...],
                              precision=_HIGH,
                              preferred_element_type=jnp.float32)
                    + b_ref[...] + h)


def _combine_final_body(s_ref, dg_ref, h_ref, xa_ref, wr_ref, b_ref, out_ref):
    h = h_ref[...]
    out = ((s_ref[0] + s_ref[1]) * _inv_deg(dg_ref)
           + jnp.dot(jnp.maximum(h, 0.0), wr_ref[...], precision=_HIGH,
                     preferred_element_type=jnp.float32)
           + b_ref[...] + h)
    out_ref[...] = (out + xa_ref[...]) * 0.5


_BS_S = pl.BlockSpec((NC, R, D), lambda i: (0, i, 0))
_BS_DG = _BS_S
_BS_H = pl.BlockSpec((R, D), lambda i: (i, 0))
_BS_W = pl.BlockSpec((D, D), lambda i: (0, 0))
_BS_B = pl.BlockSpec((1, D), lambda i: (0, 0))

_ND_F32 = jax.ShapeDtypeStruct((N, D), jnp.float32)

_dual_mm = pl.pallas_call(
    _dual_mm_body,
    grid=(GRID,),
    in_specs=[_BS_H, _BS_W, _BS_W],
    out_specs=[_BS_H, _BS_H],
    out_shape=[_ND_F32, _ND_F32],
)


def _make_combine_ba(relu_g, has_res):
    return pl.pallas_call(
        functools.partial(_combine_ba_body, relu_g, has_res),
        grid=(GRID,),
        in_specs=[_BS_S, _BS_DG, _BS_H, _BS_W, _BS_B, _BS_W],
        out_specs=[_BS_H, _BS_H],
        out_shape=[_ND_F32, _ND_F32],
    )


_combine_ba_first = _make_combine_ba(False, False)
_combine_ba_mid = _make_combine_ba(True, True)

_combine_last = pl.pallas_call(
    _combine_last_body,
    grid=(GRID,),
    in_specs=[_BS_S, _BS_DG, _BS_H, _BS_W, _BS_B],
    out_specs=_BS_H,
    out_shape=_ND_F32,
)

_combine_final = pl.pallas_call(
    _combine_final_body,
    grid=(GRID,),
    in_specs=[_BS_S, _BS_DG, _BS_H, _BS_H, _BS_W, _BS_B],
    out_specs=_BS_H,
    out_shape=_ND_F32,
)


def kernel(x, adj_t, Wl, Wr, b):
    _agg, _deg = _sc_kernels()
    src = adj_t[0].astype(jnp.int32).reshape(NW, E_PER_TILE)
    src = jnp.pad(src, ((0, 0), (0, PAD))).reshape(NW, NCH, KB)
    dst = adj_t[1].astype(jnp.int32).reshape(NW, E_PER_TILE)
    dst = jnp.pad(dst, ((0, 0), (0, PAD)), constant_values=N)
    dst = dst.reshape(NW, NCH, KB)
    zrow = jnp.zeros((NPS, D), jnp.float32)
    ones_kb = jnp.ones((KB, D), jnp.float32)
    b2 = b.reshape(-1, 1, D)

    y0, y2 = _dual_mm(x, Wl[0], Wl[2])
    deg = _deg(ones_kb, dst, zrow)
    s0 = _agg(y0, src, dst, zrow)
    h1, y1 = _combine_ba_first(s0, deg, x, Wr[0], b2[0], Wl[1])
    s1 = _agg(y1, src, dst, zrow)
    xa = _combine_last(s1, deg, h1, Wr[1], b2[1])

    s2 = _agg(y2, src, dst, zrow)
    h3, y3 = _combine_ba_first(s2, deg, x, Wr[2], b2[2], Wl[3])
    s3 = _agg(y3, src, dst, zrow)
    h4, y4 = _combine_ba_mid(s3, deg, h3, Wr[3], b2[3], Wl[4])
    s4 = _agg(y4, src, dst, zrow)
    return _combine_final(s4, deg, h4, xa, Wr[4], b2[4])


# speedup vs baseline: 1.0020x; 1.0020x over previous
import functools

import jax
import jax.numpy as jnp
from jax import lax
from jax.experimental import pallas as pl
from jax.experimental.pallas import tpu as pltpu, tpu_sc as plsc

N = 10000
E = 320000
D = 128
NC = 2
NS = 16
NW = NC * NS
E_PER_TILE = E // NW
KB = 128
NCH = 80
PAD = NCH * KB - E_PER_TILE
NACC = N + NS
NPS = 624
NREM = N - NS * NPS
ZREM = NACC - NS * NPS

_HIGH = jax.lax.Precision.HIGHEST


def _zero_acc(s, zrow_hbm, acc):
    pltpu.sync_copy(zrow_hbm, acc.at[pl.ds(s * NPS, NPS)])

    @pl.when(s == NS - 1)
    def _zero_rem():
        pltpu.sync_copy(zrow_hbm.at[pl.ds(0, ZREM)],
                        acc.at[pl.ds(NS * NPS, ZREM)])


def _readout(c, s, acc, s_out):
    pltpu.sync_copy(acc.at[pl.ds(s * NPS, NPS)],
                    s_out.at[c, pl.ds(s * NPS, NPS)])

    @pl.when(s == NS - 1)
    def _out_rem():
        pltpu.sync_copy(acc.at[pl.ds(N - NREM, NREM)],
                        s_out.at[c, pl.ds(N - NREM, NREM)])


IDXB = NCH // 2


def _agg_body(y_hbm, src_hbm, dst_hbm, zrow_hbm, s_out,
              src_v, dst_v, rows0, acc):
    c = lax.axis_index("c")
    s = lax.axis_index("s")
    wid = c * NS + s

    _zero_acc(s, zrow_hbm, acc)
    plsc.subcore_barrier()

    for p in range(NCH // IDXB):
        pltpu.sync_copy(src_hbm.at[wid, pl.ds(p * IDXB, IDXB)], src_v)
        pltpu.sync_copy(dst_hbm.at[wid, pl.ds(p * IDXB, IDXB)], dst_v)

        def step(j, t):
            pltpu.sync_copy(y_hbm.at[src_v.at[j]], rows0)
            pltpu.sync_copy(rows0, acc.at[dst_v.at[j]], add=True)
            return t

        lax.fori_loop(0, IDXB, step, 0)

    plsc.subcore_barrier()
    _readout(c, s, acc, s_out)


def _deg_body(ones_hbm, dst_hbm, zrow_hbm, deg_out,
              dst_v, rows0, acc):
    c = lax.axis_index("c")
    s = lax.axis_index("s")
    wid = c * NS + s

    pltpu.sync_copy(ones_hbm, rows0)
    _zero_acc(s, zrow_hbm, acc)
    plsc.subcore_barrier()

    for p in range(NCH // IDXB):
        pltpu.sync_copy(dst_hbm.at[wid, pl.ds(p * IDXB, IDXB)], dst_v)

        def step(j, t):
            pltpu.sync_copy(rows0, acc.at[dst_v.at[j]], add=True)
            return t

        lax.fori_loop(0, IDXB, step, 0)

    plsc.subcore_barrier()
    _readout(c, s, acc, deg_out)


@functools.cache
def _sc_kernels():
    mesh = plsc.VectorSubcoreMesh(
        core_axis_name="c", subcore_axis_name="s",
        num_cores=NC, num_subcores=NS)
    agg = pl.kernel(
        _agg_body,
        out_type=jax.ShapeDtypeStruct((NC, N, D), jnp.float32),
        mesh=mesh,
        scratch_types=[
            pltpu.VMEM((IDXB, KB), jnp.int32),
            pltpu.VMEM((IDXB, KB), jnp.int32),
            pltpu.VMEM((KB, D), jnp.float32),
            pltpu.VMEM_SHARED((NACC, D), jnp.float32),
        ],
    )
    deg = pl.kernel(
        _deg_body,
        out_type=jax.ShapeDtypeStruct((NC, N, D), jnp.float32),
        mesh=mesh,
        scratch_types=[
            pltpu.VMEM((IDXB, KB), jnp.int32),
            pltpu.VMEM((KB, D), jnp.float32),
            pltpu.VMEM_SHARED((NACC, D), jnp.float32),
        ],
    )
    return agg, deg


R = 2000
GRID = N // R


def _dual_mm_body(x_ref, w1_ref, w2_ref, y1_ref, y2_ref):
    xb = x_ref[...]
    y1_ref[...] = jnp.dot(xb, w1_ref[...], precision=_HIGH,
                          preferred_element_type=jnp.float32)
    y2_ref[...] = jnp.dot(xb, w2_ref[...], precision=_HIGH,
                          preferred_element_type=jnp.float32)


def _inv_deg(dg_ref):
    deg = dg_ref[0] + dg_ref[1]
    return 1.0 / jnp.maximum(deg[:, 0:1], 1.0)


def _combine_ba_body(relu_g, has_res, s_ref, dg_ref, h_ref, wr_ref, b_ref,
                     wln_ref, out_ref, y_ref):
    h = h_ref[...]
    g = jnp.maximum(h, 0.0) if relu_g else h
    out = ((s_ref[0] + s_ref[1]) * _inv_deg(dg_ref)
           + jnp.dot(g, wr_ref[...], precision=_HIGH,
                     preferred_element_type=jnp.float32)
           + b_ref[...])
    if has_res:
        out = out + h
    out_ref[...] = out
    y_ref[...] = jnp.dot(jnp.maximum(out, 0.0), wln_ref[...], precision=_HIGH,
                         preferred_element_type=jnp.float32)


def _combine_last_body(s_ref, dg_ref, h_ref, wr_ref, b_ref, out_ref):
    h = h_ref[...]
    out_ref[...] = ((s_ref[0] + s_ref[1]) * _inv_deg(dg_ref)
                    + jnp.dot(jnp.maximum(h, 0.0), wr_ref[...],
                              precision=_HIGH,
                              preferred_element_type=jnp.float32)
                    + b_ref[...] + h)


def _combine_final_body(s_ref, dg_ref, h_ref, xa_ref, wr_ref, b_ref, out_ref):
    h = h_ref[...]
    out = ((s_ref[0] + s_ref[1]) * _inv_deg(dg_ref)
           + jnp.dot(jnp.maximum(h, 0.0), wr_ref[...], precision=_HIGH,
                     preferred_element_type=jnp.float32)
           + b_ref[...] + h)
    out_ref[...] = (out + xa_ref[...]) * 0.5


_BS_S = pl.BlockSpec((NC, R, D), lambda i: (0, i, 0))
_BS_DG = _BS_S
_BS_H = pl.BlockSpec((R, D), lambda i: (i, 0))
_BS_W = pl.BlockSpec((D, D), lambda i: (0, 0))
_BS_B = pl.BlockSpec((1, D), lambda i: (0, 0))

_ND_F32 = jax.ShapeDtypeStruct((N, D), jnp.float32)

_dual_mm = pl.pallas_call(
    _dual_mm_body,
    grid=(GRID,),
    in_specs=[_BS_H, _BS_W, _BS_W],
    out_specs=[_BS_H, _BS_H],
    out_shape=[_ND_F32, _ND_F32],
)


def _make_combine_ba(relu_g, has_res):
    return pl.pallas_call(
        functools.partial(_combine_ba_body, relu_g, has_res),
        grid=(GRID,),
        in_specs=[_BS_S, _BS_DG, _BS_H, _BS_W, _BS_B, _BS_W],
        out_specs=[_BS_H, _BS_H],
        out_shape=[_ND_F32, _ND_F32],
    )


_combine_ba_first = _make_combine_ba(False, False)
_combine_ba_mid = _make_combine_ba(True, True)

_combine_last = pl.pallas_call(
    _combine_last_body,
    grid=(GRID,),
    in_specs=[_BS_S, _BS_DG, _BS_H, _BS_W, _BS_B],
    out_specs=_BS_H,
    out_shape=_ND_F32,
)

_combine_final = pl.pallas_call(
    _combine_final_body,
    grid=(GRID,),
    in_specs=[_BS_S, _BS_DG, _BS_H, _BS_H, _BS_W, _BS_B],
    out_specs=_BS_H,
    out_shape=_ND_F32,
)


def kernel(x, adj_t, Wl, Wr, b):
    _agg, _deg = _sc_kernels()
    src = adj_t[0].astype(jnp.int32).reshape(NW, E_PER_TILE)
    src = jnp.pad(src, ((0, 0), (0, PAD))).reshape(NW, NCH, KB)
    dst = adj_t[1].astype(jnp.int32).reshape(NW, E_PER_TILE)
    trash = (N + jnp.arange(NW, dtype=jnp.int32) % NS)[:, None]
    dst = jnp.concatenate(
        [dst, jnp.broadcast_to(trash, (NW, PAD))], axis=1).reshape(NW, NCH, KB)
    zrow = jnp.zeros((NPS, D), jnp.float32)
    ones_kb = jnp.ones((KB, D), jnp.float32)
    b2 = b.reshape(-1, 1, D)

    y0, y2 = _dual_mm(x, Wl[0], Wl[2])
    deg = _deg(ones_kb, dst, zrow)
    s0 = _agg(y0, src, dst, zrow)
    h1, y1 = _combine_ba_first(s0, deg, x, Wr[0], b2[0], Wl[1])
    s1 = _agg(y1, src, dst, zrow)
    xa = _combine_last(s1, deg, h1, Wr[1], b2[1])

    s2 = _agg(y2, src, dst, zrow)
    h3, y3 = _combine_ba_first(s2, deg, x, Wr[2], b2[2], Wl[3])
    s3 = _agg(y3, src, dst, zrow)
    h4, y4 = _combine_ba_mid(s3, deg, h3, Wr[3], b2[3], Wl[4])
    s4 = _agg(y4, src, dst, zrow)
    return _combine_final(s4, deg, h4, xa, Wr[4], b2[4])


# speedup vs baseline: 2.0510x; 2.0468x over previous
import functools

import jax
import jax.numpy as jnp
from jax import lax
from jax.experimental import pallas as pl
from jax.experimental.pallas import tpu as pltpu, tpu_sc as plsc

N = 10000
E = 320000
D = 128
NC = 2
NS = 16
NW = NC * NS
E_PER_TILE = E // NW
KB = 80
NCH = E_PER_TILE // KB
NACC = N
NPS = 624
NREM = N - NS * NPS
ZREM = NACC - NS * NPS

_HIGH = jax.lax.Precision.HIGHEST


def _zero_acc(s, zrow_hbm, acc):
    pltpu.sync_copy(zrow_hbm, acc.at[pl.ds(s * NPS, NPS)])

    @pl.when(s == NS - 1)
    def _zero_rem():
        pltpu.sync_copy(zrow_hbm.at[pl.ds(0, ZREM)],
                        acc.at[pl.ds(NS * NPS, ZREM)])


def _readout(c, s, acc, s_out):
    pltpu.sync_copy(acc.at[pl.ds(s * NPS, NPS)],
                    s_out.at[c, pl.ds(s * NPS, NPS)])

    @pl.when(s == NS - 1)
    def _out_rem():
        pltpu.sync_copy(acc.at[pl.ds(N - NREM, NREM)],
                        s_out.at[c, pl.ds(N - NREM, NREM)])


def _agg_body(y_hbm, src_hbm, dst_hbm, zrow_hbm, s_out,
              src_v, dst_v, rows0, acc):
    c = lax.axis_index("c")
    s = lax.axis_index("s")
    wid = c * NS + s

    pltpu.sync_copy(src_hbm.at[wid], src_v)
    pltpu.sync_copy(dst_hbm.at[wid], dst_v)
    _zero_acc(s, zrow_hbm, acc)
    plsc.subcore_barrier()

    def step(j, t):
        pltpu.sync_copy(y_hbm.at[src_v.at[j]], rows0)
        pltpu.sync_copy(rows0, acc.at[dst_v.at[j]], add=True)
        return t

    lax.fori_loop(0, NCH, step, 0)
    plsc.subcore_barrier()
    _readout(c, s, acc, s_out)


def _deg_body(ones_hbm, dst_hbm, zrow_hbm, deg_out,
              dst_v, rows0, acc):
    c = lax.axis_index("c")
    s = lax.axis_index("s")
    wid = c * NS + s

    pltpu.sync_copy(dst_hbm.at[wid], dst_v)
    pltpu.sync_copy(ones_hbm, rows0)
    _zero_acc(s, zrow_hbm, acc)
    plsc.subcore_barrier()

    def step(j, t):
        pltpu.sync_copy(rows0, acc.at[dst_v.at[j]], add=True)
        return t

    lax.fori_loop(0, NCH, step, 0)
    plsc.subcore_barrier()
    _readout(c, s, acc, deg_out)


@functools.cache
def _sc_kernels():
    mesh = plsc.VectorSubcoreMesh(
        core_axis_name="c", subcore_axis_name="s",
        num_cores=NC, num_subcores=NS)
    agg = pl.kernel(
        _agg_body,
        out_type=jax.ShapeDtypeStruct((NC, N, D), jnp.float32),
        mesh=mesh,
        scratch_types=[
            pltpu.VMEM((NCH, KB), jnp.int32),
            pltpu.VMEM((NCH, KB), jnp.int32),
            pltpu.VMEM((KB, D), jnp.float32),
            pltpu.VMEM_SHARED((NACC, D), jnp.float32),
        ],
    )
    deg = pl.kernel(
        _deg_body,
        out_type=jax.ShapeDtypeStruct((NC, N, D), jnp.float32),
        mesh=mesh,
        scratch_types=[
            pltpu.VMEM((NCH, KB), jnp.int32),
            pltpu.VMEM((KB, D), jnp.float32),
            pltpu.VMEM_SHARED((NACC, D), jnp.float32),
        ],
    )
    return agg, deg


R = 2000
GRID = N // R


def _dual_mm_body(x_ref, w1_ref, w2_ref, y1_ref, y2_ref):
    xb = x_ref[...]
    y1_ref[...] = jnp.dot(xb, w1_ref[...], precision=_HIGH,
                          preferred_element_type=jnp.float32)
    y2_ref[...] = jnp.dot(xb, w2_ref[...], precision=_HIGH,
                          preferred_element_type=jnp.float32)


def _inv_deg(dg_ref):
    deg = dg_ref[0] + dg_ref[1]
    return 1.0 / jnp.maximum(deg[:, 0:1], 1.0)


def _combine_ba_body(relu_g, has_res, s_ref, dg_ref, h_ref, wr_ref, b_ref,
                     wln_ref, out_ref, y_ref):
    h = h_ref[...]
    g = jnp.maximum(h, 0.0) if relu_g else h
    out = ((s_ref[0] + s_ref[1]) * _inv_deg(dg_ref)
           + jnp.dot(g, wr_ref[...], precision=_HIGH,
                     preferred_element_type=jnp.float32)
           + b_ref[...])
    if has_res:
        out = out + h
    out_ref[...] = out
    y_ref[...] = jnp.dot(jnp.maximum(out, 0.0), wln_ref[...], precision=_HIGH,
                         preferred_element_type=jnp.float32)


def _combine_last_body(s_ref, dg_ref, h_ref, wr_ref, b_ref, out_ref):
    h = h_ref[...]
    out_ref[...] = ((s_ref[0] + s_ref[1]) * _inv_deg(dg_ref)
                    + jnp.dot(jnp.maximum(h, 0.0), wr_ref[...],
                              precision=_HIGH,
                              preferred_element_type=jnp.float32)
                    + b_ref[...] + h)


def _combine_final_body(s_ref, dg_ref, h_ref, xa_ref, wr_ref, b_ref, out_ref):
    h = h_ref[...]
    out = ((s_ref[0] + s_ref[1]) * _inv_deg(dg_ref)
           + jnp.dot(jnp.maximum(h, 0.0), wr_ref[...], precision=_HIGH,
                     preferred_element_type=jnp.float32)
           + b_ref[...] + h)
    out_ref[...] = (out + xa_ref[...]) * 0.5


_BS_S = pl.BlockSpec((NC, R, D), lambda i: (0, i, 0))
_BS_DG = _BS_S
_BS_H = pl.BlockSpec((R, D), lambda i: (i, 0))
_BS_W = pl.BlockSpec((D, D), lambda i: (0, 0))
_BS_B = pl.BlockSpec((1, D), lambda i: (0, 0))

_ND_F32 = jax.ShapeDtypeStruct((N, D), jnp.float32)

_dual_mm = pl.pallas_call(
    _dual_mm_body,
    grid=(GRID,),
    in_specs=[_BS_H, _BS_W, _BS_W],
    out_specs=[_BS_H, _BS_H],
    out_shape=[_ND_F32, _ND_F32],
)


def _make_combine_ba(relu_g, has_res):
    return pl.pallas_call(
        functools.partial(_combine_ba_body, relu_g, has_res),
        grid=(GRID,),
        in_specs=[_BS_S, _BS_DG, _BS_H, _BS_W, _BS_B, _BS_W],
        out_specs=[_BS_H, _BS_H],
        out_shape=[_ND_F32, _ND_F32],
    )


_combine_ba_first = _make_combine_ba(False, False)
_combine_ba_mid = _make_combine_ba(True, True)

_combine_last = pl.pallas_call(
    _combine_last_body,
    grid=(GRID,),
    in_specs=[_BS_S, _BS_DG, _BS_H, _BS_W, _BS_B],
    out_specs=_BS_H,
    out_shape=_ND_F32,
)

_combine_final = pl.pallas_call(
    _combine_final_body,
    grid=(GRID,),
    in_specs=[_BS_S, _BS_DG, _BS_H, _BS_H, _BS_W, _BS_B],
    out_specs=_BS_H,
    out_shape=_ND_F32,
)


def kernel(x, adj_t, Wl, Wr, b):
    _agg, _deg = _sc_kernels()
    src = adj_t[0].astype(jnp.int32).reshape(NW, NCH, KB)
    dst = adj_t[1].astype(jnp.int32).reshape(NW, NCH, KB)
    zrow = jnp.zeros((NPS, D), jnp.float32)
    ones_kb = jnp.ones((KB, D), jnp.float32)
    b2 = b.reshape(-1, 1, D)

    y0, y2 = _dual_mm(x, Wl[0], Wl[2])
    deg = _deg(ones_kb, dst, zrow)
    s0 = _agg(y0, src, dst, zrow)
    h1, y1 = _combine_ba_first(s0, deg, x, Wr[0], b2[0], Wl[1])
    s1 = _agg(y1, src, dst, zrow)
    xa = _combine_last(s1, deg, h1, Wr[1], b2[1])

    s2 = _agg(y2, src, dst, zrow)
    h3, y3 = _combine_ba_first(s2, deg, x, Wr[2], b2[2], Wl[3])
    s3 = _agg(y3, src, dst, zrow)
    h4, y4 = _combine_ba_mid(s3, deg, h3, Wr[3], b2[3], Wl[4])
    s4 = _agg(y4, src, dst, zrow)
    return _combine_final(s4, deg, h4, xa, Wr[4], b2[4])


# speedup vs baseline: 3.0408x; 1.4826x over previous
import functools

import jax
import jax.numpy as jnp
from jax import lax
from jax.experimental import pallas as pl
from jax.experimental.pallas import tpu as pltpu, tpu_sc as plsc

N = 10000
E = 320000
D = 128
NC = 2
NS = 16
NW = NC * NS
E_PER_TILE = E // NW
KB = 80
NCH = E_PER_TILE // KB
NACC = N
IDXB = 25
NPS = 624
NREM = N - NS * NPS
ZREM = NACC - NS * NPS

_HIGH = jax.lax.Precision.HIGHEST


def _zero_acc(s, zrow_hbm, acc):
    pltpu.sync_copy(zrow_hbm, acc.at[pl.ds(s * NPS, NPS)])

    @pl.when(s == NS - 1)
    def _zero_rem():
        pltpu.sync_copy(zrow_hbm.at[pl.ds(0, ZREM)],
                        acc.at[pl.ds(NS * NPS, ZREM)])


def _readout(c, s, acc, s_out):
    pltpu.sync_copy(acc.at[pl.ds(s * NPS, NPS)],
                    s_out.at[c, pl.ds(s * NPS, NPS)])

    @pl.when(s == NS - 1)
    def _out_rem():
        pltpu.sync_copy(acc.at[pl.ds(N - NREM, NREM)],
                        s_out.at[c, pl.ds(N - NREM, NREM)])


def _agg_body(y_hbm, src_hbm, dst_hbm, zrow_hbm, s_out,
              src_v, dst_v, rows0, rows1, acc, sem0, sem1):
    c = lax.axis_index("c")
    s = lax.axis_index("s")
    wid = c * NS + s

    _zero_acc(s, zrow_hbm, acc)
    plsc.subcore_barrier()

    for p in range(NCH // IDXB):
        pltpu.sync_copy(src_hbm.at[wid, p], src_v)
        pltpu.sync_copy(dst_hbm.at[wid, p], dst_v)
        pltpu.async_copy(y_hbm.at[src_v.at[0]], rows0, sem0)
        pltpu.async_copy(y_hbm.at[src_v.at[1]], rows1, sem1)

        def step(i, t):
            for off, rows, sem in ((0, rows0, sem0), (1, rows1, sem1)):
                j = 2 * i + off
                pltpu.make_async_copy(y_hbm.at[src_v.at[j]], rows, sem).wait()
                pltpu.sync_copy(rows, acc.at[dst_v.at[j]], add=True)

                @pl.when(j + 2 < IDXB)
                def _prefetch():
                    pltpu.async_copy(y_hbm.at[src_v.at[j + 2]], rows, sem)
            return t

        lax.fori_loop(0, IDXB // 2, step, 0)
        j = IDXB - 1
        pltpu.make_async_copy(y_hbm.at[src_v.at[j]], rows0, sem0).wait()
        pltpu.sync_copy(rows0, acc.at[dst_v.at[j]], add=True)

    plsc.subcore_barrier()
    _readout(c, s, acc, s_out)


def _deg_body(ones_hbm, dst_hbm, zrow_hbm, deg_out,
              dst_v, rows0, acc):
    c = lax.axis_index("c")
    s = lax.axis_index("s")
    wid = c * NS + s

    pltpu.sync_copy(ones_hbm, rows0)
    _zero_acc(s, zrow_hbm, acc)
    plsc.subcore_barrier()

    for p in range(NCH // IDXB):
        pltpu.sync_copy(dst_hbm.at[wid, p], dst_v)

        def step(j, t):
            pltpu.sync_copy(rows0, acc.at[dst_v.at[j]], add=True)
            return t

        lax.fori_loop(0, IDXB, step, 0)

    plsc.subcore_barrier()
    _readout(c, s, acc, deg_out)


@functools.cache
def _sc_kernels():
    mesh = plsc.VectorSubcoreMesh(
        core_axis_name="c", subcore_axis_name="s",
        num_cores=NC, num_subcores=NS)
    agg = pl.kernel(
        _agg_body,
        out_type=jax.ShapeDtypeStruct((NC, N, D), jnp.float32),
        mesh=mesh,
        scratch_types=[
            pltpu.VMEM((IDXB, KB), jnp.int32),
            pltpu.VMEM((IDXB, KB), jnp.int32),
            pltpu.VMEM((KB, D), jnp.float32),
            pltpu.VMEM((KB, D), jnp.float32),
            pltpu.VMEM_SHARED((NACC, D), jnp.float32),
            pltpu.SemaphoreType.DMA,
            pltpu.SemaphoreType.DMA,
        ],
    )
    deg = pl.kernel(
        _deg_body,
        out_type=jax.ShapeDtypeStruct((NC, N, D), jnp.float32),
        mesh=mesh,
        scratch_types=[
            pltpu.VMEM((IDXB, KB), jnp.int32),
            pltpu.VMEM((KB, D), jnp.float32),
            pltpu.VMEM_SHARED((NACC, D), jnp.float32),
        ],
    )
    return agg, deg


R = 2000
GRID = N // R


def _dual_mm_body(x_ref, w1_ref, w2_ref, y1_ref, y2_ref):
    xb = x_ref[...]
    y1_ref[...] = jnp.dot(xb, w1_ref[...], precision=_HIGH,
                          preferred_element_type=jnp.float32)
    y2_ref[...] = jnp.dot(xb, w2_ref[...], precision=_HIGH,
                          preferred_element_type=jnp.float32)


def _inv_deg(dg_ref):
    deg = dg_ref[0] + dg_ref[1]
    return 1.0 / jnp.maximum(deg[:, 0:1], 1.0)


def _combine_ba_body(relu_g, has_res, s_ref, dg_ref, h_ref, wr_ref, b_ref,
                     wln_ref, out_ref, y_ref):
    h = h_ref[...]
    g = jnp.maximum(h, 0.0) if relu_g else h
    out = ((s_ref[0] + s_ref[1]) * _inv_deg(dg_ref)
           + jnp.dot(g, wr_ref[...], precision=_HIGH,
                     preferred_element_type=jnp.float32)
           + b_ref[...])
    if has_res:
        out = out + h
    out_ref[...] = out
    y_ref[...] = jnp.dot(jnp.maximum(out, 0.0), wln_ref[...], precision=_HIGH,
                         preferred_element_type=jnp.float32)


def _combine_last_body(s_ref, dg_ref, h_ref, wr_ref, b_ref, out_ref):
    h = h_ref[...]
    out_ref[...] = ((s_ref[0] + s_ref[1]) * _inv_deg(dg_ref)
                    + jnp.dot(jnp.maximum(h, 0.0), wr_ref[...],
                              precision=_HIGH,
                              preferred_element_type=jnp.float32)
                    + b_ref[...] + h)


def _combine_final_body(s_ref, dg_ref, h_ref, xa_ref, wr_ref, b_ref, out_ref):
    h = h_ref[...]
    out = ((s_ref[0] + s_ref[1]) * _inv_deg(dg_ref)
           + jnp.dot(jnp.maximum(h, 0.0), wr_ref[...], precision=_HIGH,
                     preferred_element_type=jnp.float32)
           + b_ref[...] + h)
    out_ref[...] = (out + xa_ref[...]) * 0.5


_BS_S = pl.BlockSpec((NC, R, D), lambda i: (0, i, 0))
_BS_DG = _BS_S
_BS_H = pl.BlockSpec((R, D), lambda i: (i, 0))
_BS_W = pl.BlockSpec((D, D), lambda i: (0, 0))
_BS_B = pl.BlockSpec((1, D), lambda i: (0, 0))

_ND_F32 = jax.ShapeDtypeStruct((N, D), jnp.float32)

_dual_mm = pl.pallas_call(
    _dual_mm_body,
    grid=(GRID,),
    in_specs=[_BS_H, _BS_W, _BS_W],
    out_specs=[_BS_H, _BS_H],
    out_shape=[_ND_F32, _ND_F32],
)


def _make_combine_ba(relu_g, has_res):
    return pl.pallas_call(
        functools.partial(_combine_ba_body, relu_g, has_res),
        grid=(GRID,),
        in_specs=[_BS_S, _BS_DG, _BS_H, _BS_W, _BS_B, _BS_W],
        out_specs=[_BS_H, _BS_H],
        out_shape=[_ND_F32, _ND_F32],
    )


_combine_ba_first = _make_combine_ba(False, False)
_combine_ba_mid = _make_combine_ba(True, True)

_combine_last = pl.pallas_call(
    _combine_last_body,
    grid=(GRID,),
    in_specs=[_BS_S, _BS_DG, _BS_H, _BS_W, _BS_B],
    out_specs=_BS_H,
    out_shape=_ND_F32,
)

_combine_final = pl.pallas_call(
    _combine_final_body,
    grid=(GRID,),
    in_specs=[_BS_S, _BS_DG, _BS_H, _BS_H, _BS_W, _BS_B],
    out_specs=_BS_H,
    out_shape=_ND_F32,
)


def kernel(x, adj_t, Wl, Wr, b):
    _agg, _deg = _sc_kernels()
    src = adj_t[0].astype(jnp.int32).reshape(NW, NCH // IDXB, IDXB, KB)
    dst = adj_t[1].astype(jnp.int32).reshape(NW, NCH // IDXB, IDXB, KB)
    zrow = jnp.zeros((NPS, D), jnp.float32)
    ones_kb = jnp.ones((KB, D), jnp.float32)
    b2 = b.reshape(-1, 1, D)

    y0, y2 = _dual_mm(x, Wl[0], Wl[2])
    deg = _deg(ones_kb, dst, zrow)
    s0 = _agg(y0, src, dst, zrow)
    h1, y1 = _combine_ba_first(s0, deg, x, Wr[0], b2[0], Wl[1])
    s1 = _agg(y1, src, dst, zrow)
    xa = _combine_last(s1, deg, h1, Wr[1], b2[1])

    s2 = _agg(y2, src, dst, zrow)
    h3, y3 = _combine_ba_first(s2, deg, x, Wr[2], b2[2], Wl[3])
    s3 = _agg(y3, src, dst, zrow)
    h4, y4 = _combine_ba_mid(s3, deg, h3, Wr[3], b2[3], Wl[4])
    s4 = _agg(y4, src, dst, zrow)
    return _combine_final(s4, deg, h4, xa, Wr[4], b2[4])


# speedup vs baseline: 3.4235x; 1.1258x over previous
import functools

import jax
import jax.numpy as jnp
from jax import lax
from jax.experimental import pallas as pl
from jax.experimental.pallas import tpu as pltpu, tpu_sc as plsc

N = 10000
E = 320000
D = 128
NC = 2
NS = 16
NW = NC * NS
E_PER_TILE = E // NW
KB = 80
NCH = E_PER_TILE // KB
NACC = N
IDXB = 25
NPS = 624
NREM = N - NS * NPS
ZREM = NACC - NS * NPS

_HIGH = jax.lax.Precision.HIGHEST


def _zero_acc(s, zrow_hbm, acc):
    pltpu.sync_copy(zrow_hbm, acc.at[pl.ds(s * NPS, NPS)])

    @pl.when(s == NS - 1)
    def _zero_rem():
        pltpu.sync_copy(zrow_hbm.at[pl.ds(0, ZREM)],
                        acc.at[pl.ds(NS * NPS, ZREM)])


def _readout(c, s, acc, s_out):
    pltpu.sync_copy(acc.at[pl.ds(s * NPS, NPS)],
                    s_out.at[c, pl.ds(s * NPS, NPS)])

    @pl.when(s == NS - 1)
    def _out_rem():
        pltpu.sync_copy(acc.at[pl.ds(N - NREM, NREM)],
                        s_out.at[c, pl.ds(N - NREM, NREM)])


def _agg_body(y_hbm, src_hbm, dst_hbm, zrow_hbm, s_out,
              src_v, dst_v, rows0, rows1, rows2, acc, sem0, sem1, sem2):
    c = lax.axis_index("c")
    s = lax.axis_index("s")
    wid = c * NS + s

    _zero_acc(s, zrow_hbm, acc)
    plsc.subcore_barrier()

    for p in range(NCH // IDXB):
        pltpu.sync_copy(src_hbm.at[wid, p], src_v)
        pltpu.sync_copy(dst_hbm.at[wid, p], dst_v)
        pltpu.async_copy(y_hbm.at[src_v.at[0]], rows0, sem0)
        pltpu.async_copy(y_hbm.at[src_v.at[1]], rows1, sem1)
        pltpu.async_copy(y_hbm.at[src_v.at[2]], rows2, sem2)

        def step(i, t):
            for off, rows, sem in ((0, rows0, sem0), (1, rows1, sem1),
                                   (2, rows2, sem2)):
                j = 3 * i + off
                pltpu.make_async_copy(y_hbm.at[src_v.at[j]], rows, sem).wait()
                pltpu.sync_copy(rows, acc.at[dst_v.at[j]], add=True)

                @pl.when(j + 3 < IDXB)
                def _prefetch():
                    pltpu.async_copy(y_hbm.at[src_v.at[j + 3]], rows, sem)
            return t

        lax.fori_loop(0, IDXB // 3, step, 0)
        j = IDXB - 1
        pltpu.make_async_copy(y_hbm.at[src_v.at[j]], rows0, sem0).wait()
        pltpu.sync_copy(rows0, acc.at[dst_v.at[j]], add=True)

    plsc.subcore_barrier()
    _readout(c, s, acc, s_out)


def _deg_body(ones_hbm, dst_hbm, zrow_hbm, deg_out,
              dst_v, rows0, acc):
    c = lax.axis_index("c")
    s = lax.axis_index("s")
    wid = c * NS + s

    pltpu.sync_copy(ones_hbm, rows0)
    _zero_acc(s, zrow_hbm, acc)
    plsc.subcore_barrier()

    for p in range(NCH // IDXB):
        pltpu.sync_copy(dst_hbm.at[wid, p], dst_v)

        def step(j, t):
            pltpu.sync_copy(rows0, acc.at[dst_v.at[j]], add=True)
            return t

        lax.fori_loop(0, IDXB, step, 0)

    plsc.subcore_barrier()
    _readout(c, s, acc, deg_out)


@functools.cache
def _sc_kernels():
    mesh = plsc.VectorSubcoreMesh(
        core_axis_name="c", subcore_axis_name="s",
        num_cores=NC, num_subcores=NS)
    agg = pl.kernel(
        _agg_body,
        out_type=jax.ShapeDtypeStruct((NC, N, D), jnp.float32),
        mesh=mesh,
        scratch_types=[
            pltpu.VMEM((IDXB, KB), jnp.int32),
            pltpu.VMEM((IDXB, KB), jnp.int32),
            pltpu.VMEM((KB, D), jnp.float32),
            pltpu.VMEM((KB, D), jnp.float32),
            pltpu.VMEM((KB, D), jnp.float32),
            pltpu.VMEM_SHARED((NACC, D), jnp.float32),
            pltpu.SemaphoreType.DMA,
            pltpu.SemaphoreType.DMA,
            pltpu.SemaphoreType.DMA,
        ],
    )
    deg = pl.kernel(
        _deg_body,
        out_type=jax.ShapeDtypeStruct((NC, N, D), jnp.float32),
        mesh=mesh,
        scratch_types=[
            pltpu.VMEM((IDXB, KB), jnp.int32),
            pltpu.VMEM((KB, D), jnp.float32),
            pltpu.VMEM_SHARED((NACC, D), jnp.float32),
        ],
    )
    return agg, deg


R = 2000
GRID = N // R


def _dual_mm_body(x_ref, w1_ref, w2_ref, y1_ref, y2_ref):
    xb = x_ref[...]
    y1_ref[...] = jnp.dot(xb, w1_ref[...], precision=_HIGH,
                          preferred_element_type=jnp.float32)
    y2_ref[...] = jnp.dot(xb, w2_ref[...], precision=_HIGH,
                          preferred_element_type=jnp.float32)


def _inv_deg(dg_ref):
    deg = dg_ref[0] + dg_ref[1]
    return 1.0 / jnp.maximum(deg[:, 0:1], 1.0)


def _combine_ba_body(relu_g, has_res, s_ref, dg_ref, h_ref, wr_ref, b_ref,
                     wln_ref, out_ref, y_ref):
    h = h_ref[...]
    g = jnp.maximum(h, 0.0) if relu_g else h
    out = ((s_ref[0] + s_ref[1]) * _inv_deg(dg_ref)
           + jnp.dot(g, wr_ref[...], precision=_HIGH,
                     preferred_element_type=jnp.float32)
           + b_ref[...])
    if has_res:
        out = out + h
    out_ref[...] = out
    y_ref[...] = jnp.dot(jnp.maximum(out, 0.0), wln_ref[...], precision=_HIGH,
                         preferred_element_type=jnp.float32)


def _combine_last_body(s_ref, dg_ref, h_ref, wr_ref, b_ref, out_ref):
    h = h_ref[...]
    out_ref[...] = ((s_ref[0] + s_ref[1]) * _inv_deg(dg_ref)
                    + jnp.dot(jnp.maximum(h, 0.0), wr_ref[...],
                              precision=_HIGH,
                              preferred_element_type=jnp.float32)
                    + b_ref[...] + h)


def _combine_final_body(s_ref, dg_ref, h_ref, xa_ref, wr_ref, b_ref, out_ref):
    h = h_ref[...]
    out = ((s_ref[0] + s_ref[1]) * _inv_deg(dg_ref)
           + jnp.dot(jnp.maximum(h, 0.0), wr_ref[...], precision=_HIGH,
                     preferred_element_type=jnp.float32)
           + b_ref[...] + h)
    out_ref[...] = (out + xa_ref[...]) * 0.5


_BS_S = pl.BlockSpec((NC, R, D), lambda i: (0, i, 0))
_BS_DG = _BS_S
_BS_H = pl.BlockSpec((R, D), lambda i: (i, 0))
_BS_W = pl.BlockSpec((D, D), lambda i: (0, 0))
_BS_B = pl.BlockSpec((1, D), lambda i: (0, 0))

_ND_F32 = jax.ShapeDtypeStruct((N, D), jnp.float32)

_dual_mm = pl.pallas_call(
    _dual_mm_body,
    grid=(GRID,),
    in_specs=[_BS_H, _BS_W, _BS_W],
    out_specs=[_BS_H, _BS_H],
    out_shape=[_ND_F32, _ND_F32],
)


def _make_combine_ba(relu_g, has_res):
    return pl.pallas_call(
        functools.partial(_combine_ba_body, relu_g, has_res),
        grid=(GRID,),
        in_specs=[_BS_S, _BS_DG, _BS_H, _BS_W, _BS_B, _BS_W],
        out_specs=[_BS_H, _BS_H],
        out_shape=[_ND_F32, _ND_F32],
    )


_combine_ba_first = _make_combine_ba(False, False)
_combine_ba_mid = _make_combine_ba(True, True)

_combine_last = pl.pallas_call(
    _combine_last_body,
    grid=(GRID,),
    in_specs=[_BS_S, _BS_DG, _BS_H, _BS_W, _BS_B],
    out_specs=_BS_H,
    out_shape=_ND_F32,
)

_combine_final = pl.pallas_call(
    _combine_final_body,
    grid=(GRID,),
    in_specs=[_BS_S, _BS_DG, _BS_H, _BS_H, _BS_W, _BS_B],
    out_specs=_BS_H,
    out_shape=_ND_F32,
)


def kernel(x, adj_t, Wl, Wr, b):
    _agg, _deg = _sc_kernels()
    src = adj_t[0].astype(jnp.int32).reshape(NW, NCH // IDXB, IDXB, KB)
    dst = adj_t[1].astype(jnp.int32).reshape(NW, NCH // IDXB, IDXB, KB)
    zrow = jnp.zeros((NPS, D), jnp.float32)
    ones_kb = jnp.ones((KB, D), jnp.float32)
    b2 = b.reshape(-1, 1, D)

    y0, y2 = _dual_mm(x, Wl[0], Wl[2])
    deg = _deg(ones_kb, dst, zrow)
    s0 = _agg(y0, src, dst, zrow)
    h1, y1 = _combine_ba_first(s0, deg, x, Wr[0], b2[0], Wl[1])
    s1 = _agg(y1, src, dst, zrow)
    xa = _combine_last(s1, deg, h1, Wr[1], b2[1])

    s2 = _agg(y2, src, dst, zrow)
    h3, y3 = _combine_ba_first(s2, deg, x, Wr[2], b2[2], Wl[3])
    s3 = _agg(y3, src, dst, zrow)
    h4, y4 = _combine_ba_mid(s3, deg, h3, Wr[3], b2[3], Wl[4])
    s4 = _agg(y4, src, dst, zrow)
    return _combine_final(s4, deg, h4, xa, Wr[4], b2[4])
